# baseline (speedup 1.0000x reference)
import functools

import jax
import jax.numpy as jnp
from jax import lax
from jax.experimental import pallas as pl
from jax.experimental.pallas import tpu as pltpu
from jax.experimental.pallas import tpu_sc as plsc

N = 10000
E = 320000
C = 128
R = E + N
NC, NS, L = 2, 16, 16
NW = NC * NS
N_PAD = 10240
STRIPE = N_PAD // NS
RT = 10400
R_PAD = RT * NW
K1 = 104
ITERS1 = RT // K1
K3 = 400
ITERS3 = RT // K3
FAST_CORE = 1
PAIR_RT = 2 * RT
RT1_F, RT1_S = 13520, 7280
RT3_F, RT3_S = 13600, 7200
KC = 208
ITERSC = RT // KC
BR = 2000
NBLK = R // BR
EPS = 1e-5

_mesh = plsc.VectorSubcoreMesh(
    core_axis_name="c", subcore_axis_name="s", num_cores=NC, num_subcores=NS)

f32 = jnp.float32
i32 = jnp.int32


@functools.partial(
    pl.kernel,
    out_type=jax.ShapeDtypeStruct((NC, N_PAD, C), f32),
    mesh=_mesh,
    scratch_types=[
        pltpu.VMEM_SHARED((N_PAD, C), f32),
        pltpu.VMEM((K1,), i32),
        pltpu.VMEM((K1,), i32),
        pltpu.VMEM((K1,), i32),
        pltpu.VMEM((K1,), i32),
        pltpu.VMEM((K1, C), f32),
        pltpu.VMEM((K1, C), f32),
        [pltpu.SemaphoreType.DMA] * 2,
        [pltpu.SemaphoreType.DMA] * 2,
    ],
)
def _sc_nsum(x_hbm, src_hbm, dst_hbm, zrow_hbm,
             nsum_out, nsum_sh, sidx0, sidx1, didx0, didx1, rows0, rows1,
             semA, semW):
  sidx = [sidx0, sidx1]
  didx = [didx0, didx1]
  rows = [rows0, rows1]
  c = lax.axis_index("c")
  s = lax.axis_index("s")
  pltpu.sync_copy(zrow_hbm, nsum_sh.at[pl.ds(s * STRIPE, STRIPE), :])
  plsc.subcore_barrier()

  fast = c == FAST_CORE
  base = s * PAIR_RT + jnp.where(fast, 0, RT1_F)
  iters = jnp.where(fast, RT1_F // K1, RT1_S // K1)

  def load_and_gather(i, b):
    off = pl.multiple_of(base + i * K1, 8)
    pltpu.sync_copy(src_hbm.at[pl.ds(off, K1)], sidx[b])
    pltpu.sync_copy(dst_hbm.at[pl.ds(off, K1)], didx[b])
    pltpu.async_copy(x_hbm.at[sidx[b]], rows[b], semA[b])

  load_and_gather(0, 0)
  load_and_gather(1, 1)

  def step(t, carry):
    for b in range(2):
      i = t * 2 + b
      pltpu.make_async_copy(x_hbm.at[sidx[b]], rows[b], semA[b]).wait()
      pltpu.async_copy(rows[b], nsum_sh.at[didx[b]], semW[b], add=True)
      pltpu.make_async_copy(rows[b], nsum_sh.at[didx[b]], semW[b]).wait()

      @pl.when(i + 2 < iters)
      def _():
        load_and_gather(i + 2, b)
    return carry

  lax.fori_loop(0, iters // 2, step, 0)
  plsc.subcore_barrier()
  sl = pl.ds(s * STRIPE, STRIPE)
  pltpu.sync_copy(nsum_sh.at[sl, :], nsum_out.at[c, sl, :])


@functools.partial(
    pl.kernel,
    out_type=jax.ShapeDtypeStruct((NC, N_PAD, C), f32),
    mesh=_mesh,
    scratch_types=[
        pltpu.VMEM_SHARED((N_PAD, C), f32),
        pltpu.VMEM((KC,), i32),
        pltpu.VMEM((KC,), i32),
        pltpu.VMEM((KC, C), f32),
        [pltpu.SemaphoreType.DMA] * 2,
        [pltpu.SemaphoreType.DMA] * 2,
    ],
)
def _sc_cnt(idx_hbm, ones_hbm, zrow_hbm, acc_out, acc_sh,
            didx0, didx1, ones_v, semI, semW):
  didx = [didx0, didx1]
  c = lax.axis_index("c")
  s = lax.axis_index("s")
  wid = s * NC + c
  pltpu.sync_copy(zrow_hbm, acc_sh.at[pl.ds(s * STRIPE, STRIPE), :])
  pltpu.sync_copy(ones_hbm, ones_v)
  plsc.subcore_barrier()

  base = wid * RT

  def load_idx(i, b):
    off = pl.multiple_of(base + i * KC, 8)
    pltpu.async_copy(idx_hbm.at[pl.ds(off, KC)], didx[b], semI[b])

  load_idx(0, 0)
  load_idx(1, 1)

  def step(t, carry):
    for b in range(2):
      i = t * 2 + b
      off = pl.multiple_of(base + i * KC, 8)
      pltpu.make_async_copy(
          idx_hbm.at[pl.ds(off, KC)], didx[b], semI[b]).wait()
      pltpu.async_copy(ones_v, acc_sh.at[didx[b]], semW[b], add=True)
      pltpu.make_async_copy(ones_v, acc_sh.at[didx[b]], semW[b]).wait()

      @pl.when(i + 2 < ITERSC)
      def _():
        load_idx(i + 2, b)
    return carry

  lax.fori_loop(0, ITERSC // 2, step, 0)
  plsc.subcore_barrier()
  sl = pl.ds(s * STRIPE, STRIPE)
  pltpu.sync_copy(acc_sh.at[sl, :], acc_out.at[c, sl, :])


@functools.partial(
    pl.kernel,
    out_type=jax.ShapeDtypeStruct((R_PAD, C), f32),
    mesh=_mesh,
    scratch_types=[
        pltpu.VMEM((RT3_F,), i32),
        pltpu.VMEM((RT3_F,), i32),
        pltpu.VMEM((K3, C), f32),
        pltpu.VMEM((K3, C), f32),
        [pltpu.SemaphoreType.DMA] * 2,
        [pltpu.SemaphoreType.DMA] * 2,
        [pltpu.SemaphoreType.DMA] * 2,
    ],
)
def _sc_gather(a_hbm, b_hbm, src_hbm, dst_hbm, p_out,
               sidx, didx, pbuf0, pbuf1, semA, semB, semS):
  pbuf = [pbuf0, pbuf1]
  c = lax.axis_index("c")
  s = lax.axis_index("s")
  fast = c == FAST_CORE
  base = pl.multiple_of(s * PAIR_RT + jnp.where(fast, 0, RT3_F), 8)
  iters = jnp.where(fast, RT3_F // K3, RT3_S // K3)
  pltpu.sync_copy(src_hbm.at[pl.ds(base, RT3_S)], sidx.at[pl.ds(0, RT3_S)])
  pltpu.sync_copy(dst_hbm.at[pl.ds(base, RT3_S)], didx.at[pl.ds(0, RT3_S)])

  @pl.when(fast)
  def _():
    tail = pl.multiple_of(base + RT3_S, 8)
    pltpu.sync_copy(src_hbm.at[pl.ds(tail, RT3_F - RT3_S)],
                    sidx.at[pl.ds(RT3_S, RT3_F - RT3_S)])
    pltpu.sync_copy(dst_hbm.at[pl.ds(tail, RT3_F - RT3_S)],
                    didx.at[pl.ds(RT3_S, RT3_F - RT3_S)])

  def gather_a(i, b):
    pltpu.async_copy(a_hbm.at[sidx.at[pl.ds(i * K3, K3)]], pbuf[b], semA[b])

  gather_a(0, 0)
  gather_a(1, 1)

  def step(t, carry):
    for b in range(2):
      i = t * 2 + b
      isl = pl.ds(i * K3, K3)
      pltpu.make_async_copy(
          a_hbm.at[sidx.at[isl]], pbuf[b], semA[b]).wait()
      pltpu.async_copy(b_hbm.at[didx.at[isl]], pbuf[b], semB[b], add=True)
      pltpu.make_async_copy(
          b_hbm.at[didx.at[isl]], pbuf[b], semB[b]).wait()
      off = pl.multiple_of(base + i * K3, 8)
      pltpu.async_copy(pbuf[b], p_out.at[pl.ds(off, K3), :], semS[b])

      @pl.when(i + 2 < iters)
      def _():
        pltpu.make_async_copy(
            pbuf[b], p_out.at[pl.ds(off, K3), :], semS[b]).wait()
        gather_a(i + 2, b)
    return carry

  lax.fori_loop(0, iters // 2, step, 0)
  for b in range(2):
    i = iters - 2 + b
    off = pl.multiple_of(base + i * K3, 8)
    pltpu.make_async_copy(pbuf[b], p_out.at[pl.ds(off, K3), :], semS[b]).wait()


def _dot(a, b):
  return lax.dot_general(a, b, (((1,), (0,)), ((), ())),
                         preferred_element_type=f32)


def _fold_body(nsum_p_ref, cs_ref, cd_ref, x_ref, g1_ref, bb1_ref, W1_ref,
               b1_ref, a_ref, b_ref):
  nsum = nsum_p_ref[0] + nsum_p_ref[1]
  cs = cs_ref[0] + cs_ref[1]
  cd = cd_ref[0] + cd_ref[1]
  x = x_ref[...]
  Rf = f32(R)
  sum_g = jnp.sum(cs * x, axis=0, keepdims=True)
  sumsq_g = jnp.sum(cs * (x * x), axis=0, keepdims=True)
  sum_n = jnp.sum(cd * nsum, axis=0, keepdims=True)
  sumsq_n = jnp.sum(cd * (nsum * nsum), axis=0, keepdims=True)
  m_g = sum_g / Rf
  v_g = sumsq_g / Rf - m_g * m_g
  m_n = sum_n / Rf
  v_n = sumsq_n / Rf - m_n * m_n
  g1 = g1_ref[...]
  bb1 = bb1_ref[...]
  s_g = g1[:, :C] * lax.rsqrt(v_g + EPS)
  s_n = g1[:, C:] * lax.rsqrt(v_n + EPS)
  c_g = bb1[:, :C] - m_g * s_g
  c_n = bb1[:, C:] - m_n * s_n
  W1t = W1_ref[:C, :]
  W1b = W1_ref[C:, :]
  b1p = b1_ref[...] + _dot(c_g, W1t) + _dot(c_n, W1b)
  a_ref[...] = _dot(x * s_g, W1t) + b1p
  b_ref[...] = _dot(nsum * s_n, W1b)


_tc_fold = pl.pallas_call(
    _fold_body,
    out_shape=(jax.ShapeDtypeStruct((N_PAD, C), f32),
               jax.ShapeDtypeStruct((N_PAD, C), f32)),
)


def _stats_body(p_ref, o_ref):
  i = pl.program_id(0)

  @pl.when(i == 0)
  def _():
    o_ref[...] = jnp.zeros_like(o_ref)

  h = jnp.maximum(p_ref[...], 0.0)
  o_ref[0:1, :] += jnp.sum(h, axis=0, keepdims=True)
  o_ref[1:2, :] += jnp.sum(h * h, axis=0, keepdims=True)


_tc_stats = pl.pallas_call(
    _stats_body,
    grid=(NBLK,),
    in_specs=[pl.BlockSpec((BR, C), lambda i: (i, 0))],
    out_specs=pl.BlockSpec((2, C), lambda i: (0, 0)),
    out_shape=jax.ShapeDtypeStruct((2, C), f32),
)


def _final_body(p_ref, st_ref, g2_ref, bb2_ref, W2_ref, b2_ref, o_ref):
  Rf = f32(R)
  m2 = st_ref[0:1, :] / Rf
  v2 = st_ref[1:2, :] / Rf - m2 * m2
  s2 = g2_ref[...] * lax.rsqrt(v2 + EPS)
  c2 = bb2_ref[...] - m2 * s2
  h = jnp.maximum(p_ref[...], 0.0)
  hn = h * s2 + c2
  o_ref[...] = jnp.maximum(_dot(hn, W2_ref[...]) + b2_ref[...], 0.0)


_tc_final = pl.pallas_call(
    _final_body,
    grid=(NBLK,),
    in_specs=[
        pl.BlockSpec((BR, C), lambda i: (i, 0)),
        pl.BlockSpec((2, C), lambda i: (0, 0)),
        pl.BlockSpec((1, C), lambda i: (0, 0)),
        pl.BlockSpec((1, C), lambda i: (0, 0)),
        pl.BlockSpec((C, C), lambda i: (0, 0)),
        pl.BlockSpec((1, C), lambda i: (0, 0)),
    ],
    out_specs=pl.BlockSpec((BR, C), lambda i: (i, 0)),
    out_shape=jax.ShapeDtypeStruct((R, C), f32),
)


@jax.jit
def kernel(x, edge_index, bn1_g, bn1_b, W1, b1, bn2_g, bn2_b, W2, b2):
  src = edge_index[0].astype(i32)
  dst = edge_index[1].astype(i32)
  self_idx = jnp.arange(N, dtype=i32)
  padv = jnp.full((R_PAD - R,), N, i32)
  src_all = jnp.concatenate([src, self_idx, padv])
  dst_all = jnp.concatenate([dst, self_idx, padv])
  x_pad = jnp.pad(x, ((0, N_PAD - N), (0, 0)))

  zrow = jnp.zeros((STRIPE, C), f32)
  ones_h = jnp.ones((KC, C), f32)

  nsum_parts = _sc_nsum(x_pad, src_all, dst_all, zrow)
  cs_parts = _sc_cnt(src_all, ones_h, zrow)
  cd_parts = _sc_cnt(dst_all, ones_h, zrow)
  cs = cs_parts[:, :, 0:1]
  cd = cd_parts[:, :, 0:1]

  A_pad, B_pad = _tc_fold(
      nsum_parts, cs, cd, x_pad,
      bn1_g.reshape(1, 2 * C), bn1_b.reshape(1, 2 * C),
      W1, b1.reshape(1, C))

  P = _sc_gather(A_pad, B_pad, src_all, dst_all)

  stats = _tc_stats(P)

  out = _tc_final(P, stats, bn2_g.reshape(1, C), bn2_b.reshape(1, C),
                  W2, b2.reshape(1, C))
  return out


# speedup vs baseline: 1.0064x; 1.0064x over previous
import functools

import jax
import jax.numpy as jnp
from jax import lax
from jax.experimental import pallas as pl
from jax.experimental.pallas import tpu as pltpu
from jax.experimental.pallas import tpu_sc as plsc

N = 10000
E = 320000
C = 128
R = E + N
NC, NS, L = 2, 16, 16
NW = NC * NS
N_PAD = 10240
STRIPE = N_PAD // NS
RT = 10400
R_PAD = RT * NW
K1 = 104
ITERS1 = RT // K1
K3 = 400
ITERS3 = RT // K3
FAST_CORE = 0
PAIR_RT = 2 * RT
RT1_F, RT1_S = 13520, 7280
RT3_F, RT3_S = 13600, 7200
KC = 208
ITERSC = RT // KC
BR = 2000
NBLK = R // BR
EPS = 1e-5

_mesh = plsc.VectorSubcoreMesh(
    core_axis_name="c", subcore_axis_name="s", num_cores=NC, num_subcores=NS)

f32 = jnp.float32
i32 = jnp.int32


@functools.partial(
    pl.kernel,
    out_type=jax.ShapeDtypeStruct((NC, N_PAD, C), f32),
    mesh=_mesh,
    scratch_types=[
        pltpu.VMEM_SHARED((N_PAD, C), f32),
        pltpu.VMEM((K1,), i32),
        pltpu.VMEM((K1,), i32),
        pltpu.VMEM((K1,), i32),
        pltpu.VMEM((K1,), i32),
        pltpu.VMEM((K1, C), f32),
        pltpu.VMEM((K1, C), f32),
        [pltpu.SemaphoreType.DMA] * 2,
        [pltpu.SemaphoreType.DMA] * 2,
    ],
)
def _sc_nsum(x_hbm, src_hbm, dst_hbm, zrow_hbm,
             nsum_out, nsum_sh, sidx0, sidx1, didx0, didx1, rows0, rows1,
             semA, semW):
  sidx = [sidx0, sidx1]
  didx = [didx0, didx1]
  rows = [rows0, rows1]
  c = lax.axis_index("c")
  s = lax.axis_index("s")
  pltpu.sync_copy(zrow_hbm, nsum_sh.at[pl.ds(s * STRIPE, STRIPE), :])
  plsc.subcore_barrier()

  fast = c == FAST_CORE
  base = s * PAIR_RT + jnp.where(fast, 0, RT1_F)
  iters = jnp.where(fast, RT1_F // K1, RT1_S // K1)

  def load_and_gather(i, b):
    off = pl.multiple_of(base + i * K1, 8)
    pltpu.sync_copy(src_hbm.at[pl.ds(off, K1)], sidx[b])
    pltpu.sync_copy(dst_hbm.at[pl.ds(off, K1)], didx[b])
    pltpu.async_copy(x_hbm.at[sidx[b]], rows[b], semA[b])

  load_and_gather(0, 0)
  load_and_gather(1, 1)

  def step(t, carry):
    for b in range(2):
      i = t * 2 + b
      pltpu.make_async_copy(x_hbm.at[sidx[b]], rows[b], semA[b]).wait()
      pltpu.async_copy(rows[b], nsum_sh.at[didx[b]], semW[b], add=True)
      pltpu.make_async_copy(rows[b], nsum_sh.at[didx[b]], semW[b]).wait()

      @pl.when(i + 2 < iters)
      def _():
        load_and_gather(i + 2, b)
    return carry

  lax.fori_loop(0, iters // 2, step, 0)
  plsc.subcore_barrier()
  sl = pl.ds(s * STRIPE, STRIPE)
  pltpu.sync_copy(nsum_sh.at[sl, :], nsum_out.at[c, sl, :])


@functools.partial(
    pl.kernel,
    out_type=jax.ShapeDtypeStruct((NC, N_PAD, C), f32),
    mesh=_mesh,
    scratch_types=[
        pltpu.VMEM_SHARED((N_PAD, C), f32),
        pltpu.VMEM((KC,), i32),
        pltpu.VMEM((KC,), i32),
        pltpu.VMEM((KC, C), f32),
        [pltpu.SemaphoreType.DMA] * 2,
        [pltpu.SemaphoreType.DMA] * 2,
    ],
)
def _sc_cnt(idx_hbm, ones_hbm, zrow_hbm, acc_out, acc_sh,
            didx0, didx1, ones_v, semI, semW):
  didx = [didx0, didx1]
  c = lax.axis_index("c")
  s = lax.axis_index("s")
  wid = s * NC + c
  pltpu.sync_copy(zrow_hbm, acc_sh.at[pl.ds(s * STRIPE, STRIPE), :])
  pltpu.sync_copy(ones_hbm, ones_v)
  plsc.subcore_barrier()

  base = wid * RT

  def load_idx(i, b):
    off = pl.multiple_of(base + i * KC, 8)
    pltpu.async_copy(idx_hbm.at[pl.ds(off, KC)], didx[b], semI[b])

  load_idx(0, 0)
  load_idx(1, 1)

  def step(t, carry):
    for b in range(2):
      i = t * 2 + b
      off = pl.multiple_of(base + i * KC, 8)
      pltpu.make_async_copy(
          idx_hbm.at[pl.ds(off, KC)], didx[b], semI[b]).wait()
      pltpu.async_copy(ones_v, acc_sh.at[didx[b]], semW[b], add=True)
      pltpu.make_async_copy(ones_v, acc_sh.at[didx[b]], semW[b]).wait()

      @pl.when(i + 2 < ITERSC)
      def _():
        load_idx(i + 2, b)
    return carry

  lax.fori_loop(0, ITERSC // 2, step, 0)
  plsc.subcore_barrier()
  sl = pl.ds(s * STRIPE, STRIPE)
  pltpu.sync_copy(acc_sh.at[sl, :], acc_out.at[c, sl, :])


@functools.partial(
    pl.kernel,
    out_type=jax.ShapeDtypeStruct((R_PAD, C), f32),
    mesh=_mesh,
    scratch_types=[
        pltpu.VMEM((RT3_F,), i32),
        pltpu.VMEM((RT3_F,), i32),
        pltpu.VMEM((K3, C), f32),
        pltpu.VMEM((K3, C), f32),
        [pltpu.SemaphoreType.DMA] * 2,
        [pltpu.SemaphoreType.DMA] * 2,
        [pltpu.SemaphoreType.DMA] * 2,
    ],
)
def _sc_gather(a_hbm, b_hbm, src_hbm, dst_hbm, p_out,
               sidx, didx, pbuf0, pbuf1, semA, semB, semS):
  pbuf = [pbuf0, pbuf1]
  c = lax.axis_index("c")
  s = lax.axis_index("s")
  fast = c == FAST_CORE
  base = pl.multiple_of(s * PAIR_RT + jnp.where(fast, 0, RT3_F), 8)
  iters = jnp.where(fast, RT3_F // K3, RT3_S // K3)
  pltpu.sync_copy(src_hbm.at[pl.ds(base, RT3_S)], sidx.at[pl.ds(0, RT3_S)])
  pltpu.sync_copy(dst_hbm.at[pl.ds(base, RT3_S)], didx.at[pl.ds(0, RT3_S)])

  @pl.when(fast)
  def _():
    tail = pl.multiple_of(base + RT3_S, 8)
    pltpu.sync_copy(src_hbm.at[pl.ds(tail, RT3_F - RT3_S)],
                    sidx.at[pl.ds(RT3_S, RT3_F - RT3_S)])
    pltpu.sync_copy(dst_hbm.at[pl.ds(tail, RT3_F - RT3_S)],
                    didx.at[pl.ds(RT3_S, RT3_F - RT3_S)])

  def gather_a(i, b):
    pltpu.async_copy(a_hbm.at[sidx.at[pl.ds(i * K3, K3)]], pbuf[b], semA[b])

  gather_a(0, 0)
  gather_a(1, 1)

  def step(t, carry):
    for b in range(2):
      i = t * 2 + b
      isl = pl.ds(i * K3, K3)
      pltpu.make_async_copy(
          a_hbm.at[sidx.at[isl]], pbuf[b], semA[b]).wait()
      pltpu.async_copy(b_hbm.at[didx.at[isl]], pbuf[b], semB[b], add=True)
      pltpu.make_async_copy(
          b_hbm.at[didx.at[isl]], pbuf[b], semB[b]).wait()
      off = pl.multiple_of(base + i * K3, 8)
      pltpu.async_copy(pbuf[b], p_out.at[pl.ds(off, K3), :], semS[b])

      @pl.when(i + 2 < iters)
      def _():
        pltpu.make_async_copy(
            pbuf[b], p_out.at[pl.ds(off, K3), :], semS[b]).wait()
        gather_a(i + 2, b)
    return carry

  lax.fori_loop(0, iters // 2, step, 0)
  for b in range(2):
    i = iters - 2 + b
    off = pl.multiple_of(base + i * K3, 8)
    pltpu.make_async_copy(pbuf[b], p_out.at[pl.ds(off, K3), :], semS[b]).wait()


def _dot(a, b):
  return lax.dot_general(a, b, (((1,), (0,)), ((), ())),
                         preferred_element_type=f32)


def _fold_body(nsum_p_ref, cs_ref, cd_ref, x_ref, g1_ref, bb1_ref, W1_ref,
               b1_ref, a_ref, b_ref):
  nsum = nsum_p_ref[0] + nsum_p_ref[1]
  cs = cs_ref[0] + cs_ref[1]
  cd = cd_ref[0] + cd_ref[1]
  x = x_ref[...]
  Rf = f32(R)
  sum_g = jnp.sum(cs * x, axis=0, keepdims=True)
  sumsq_g = jnp.sum(cs * (x * x), axis=0, keepdims=True)
  sum_n = jnp.sum(cd * nsum, axis=0, keepdims=True)
  sumsq_n = jnp.sum(cd * (nsum * nsum), axis=0, keepdims=True)
  m_g = sum_g / Rf
  v_g = sumsq_g / Rf - m_g * m_g
  m_n = sum_n / Rf
  v_n = sumsq_n / Rf - m_n * m_n
  g1 = g1_ref[...]
  bb1 = bb1_ref[...]
  s_g = g1[:, :C] * lax.rsqrt(v_g + EPS)
  s_n = g1[:, C:] * lax.rsqrt(v_n + EPS)
  c_g = bb1[:, :C] - m_g * s_g
  c_n = bb1[:, C:] - m_n * s_n
  W1t = W1_ref[:C, :]
  W1b = W1_ref[C:, :]
  b1p = b1_ref[...] + _dot(c_g, W1t) + _dot(c_n, W1b)
  a_ref[...] = _dot(x * s_g, W1t) + b1p
  b_ref[...] = _dot(nsum * s_n, W1b)


_tc_fold = pl.pallas_call(
    _fold_body,
    out_shape=(jax.ShapeDtypeStruct((N_PAD, C), f32),
               jax.ShapeDtypeStruct((N_PAD, C), f32)),
)


def _stats_body(p_ref, o_ref):
  i = pl.program_id(0)

  @pl.when(i == 0)
  def _():
    o_ref[...] = jnp.zeros_like(o_ref)

  h = jnp.maximum(p_ref[...], 0.0)
  o_ref[0:1, :] += jnp.sum(h, axis=0, keepdims=True)
  o_ref[1:2, :] += jnp.sum(h * h, axis=0, keepdims=True)


_tc_stats = pl.pallas_call(
    _stats_body,
    grid=(NBLK,),
    in_specs=[pl.BlockSpec((BR, C), lambda i: (i, 0))],
    out_specs=pl.BlockSpec((2, C), lambda i: (0, 0)),
    out_shape=jax.ShapeDtypeStruct((2, C), f32),
)


def _final_body(p_ref, st_ref, g2_ref, bb2_ref, W2_ref, b2_ref, o_ref):
  Rf = f32(R)
  m2 = st_ref[0:1, :] / Rf
  v2 = st_ref[1:2, :] / Rf - m2 * m2
  s2 = g2_ref[...] * lax.rsqrt(v2 + EPS)
  c2 = bb2_ref[...] - m2 * s2
  h = jnp.maximum(p_ref[...], 0.0)
  hn = h * s2 + c2
  o_ref[...] = jnp.maximum(_dot(hn, W2_ref[...]) + b2_ref[...], 0.0)


_tc_final = pl.pallas_call(
    _final_body,
    grid=(NBLK,),
    in_specs=[
        pl.BlockSpec((BR, C), lambda i: (i, 0)),
        pl.BlockSpec((2, C), lambda i: (0, 0)),
        pl.BlockSpec((1, C), lambda i: (0, 0)),
        pl.BlockSpec((1, C), lambda i: (0, 0)),
        pl.BlockSpec((C, C), lambda i: (0, 0)),
        pl.BlockSpec((1, C), lambda i: (0, 0)),
    ],
    out_specs=pl.BlockSpec((BR, C), lambda i: (i, 0)),
    out_shape=jax.ShapeDtypeStruct((R, C), f32),
)


@jax.jit
def kernel(x, edge_index, bn1_g, bn1_b, W1, b1, bn2_g, bn2_b, W2, b2):
  src = edge_index[0].astype(i32)
  dst = edge_index[1].astype(i32)
  self_idx = jnp.arange(N, dtype=i32)
  padv = jnp.full((R_PAD - R,), N, i32)
  src_all = jnp.concatenate([src, self_idx, padv])
  dst_all = jnp.concatenate([dst, self_idx, padv])
  x_pad = jnp.pad(x, ((0, N_PAD - N), (0, 0)))

  zrow = jnp.zeros((STRIPE, C), f32)
  ones_h = jnp.ones((KC, C), f32)

  nsum_parts = _sc_nsum(x_pad, src_all, dst_all, zrow)
  cs_parts = _sc_cnt(src_all, ones_h, zrow)
  cd_parts = _sc_cnt(dst_all, ones_h, zrow)
  cs = cs_parts[:, :, 0:1]
  cd = cd_parts[:, :, 0:1]

  A_pad, B_pad = _tc_fold(
      nsum_parts, cs, cd, x_pad,
      bn1_g.reshape(1, 2 * C), bn1_b.reshape(1, 2 * C),
      W1, b1.reshape(1, C))

  P = _sc_gather(A_pad, B_pad, src_all, dst_all)

  stats = _tc_stats(P)

  out = _tc_final(P, stats, bn2_g.reshape(1, C), bn2_b.reshape(1, C),
                  W2, b2.reshape(1, C))
  return out


# speedup vs baseline: 1.0180x; 1.0116x over previous
import functools

import jax
import jax.numpy as jnp
from jax import lax
from jax.experimental import pallas as pl
from jax.experimental.pallas import tpu as pltpu
from jax.experimental.pallas import tpu_sc as plsc

N = 10000
E = 320000
C = 128
R = E + N
NC, NS, L = 2, 16, 16
NW = NC * NS
N_PAD = 10240
STRIPE = N_PAD // NS
RT = 10400
R_PAD = RT * NW
K1 = 104
ITERS1 = RT // K1
K3 = 400
ITERS3 = RT // K3
FAST_CORE = 0
PAIR_RT = 2 * RT
RT1_F, RT1_S = 13520, 7280
RT3_F, RT3_S = 13600, 7200
KC = 208
ITERSC = RT // KC
BR = 2000
NBLK = R // BR
EPS = 1e-5

_mesh = plsc.VectorSubcoreMesh(
    core_axis_name="c", subcore_axis_name="s", num_cores=NC, num_subcores=NS)

f32 = jnp.float32
i32 = jnp.int32


@functools.partial(
    pl.kernel,
    out_type=jax.ShapeDtypeStruct((NC, N_PAD, C), f32),
    mesh=_mesh,
    scratch_types=[
        pltpu.VMEM_SHARED((N_PAD, C), f32),
        pltpu.VMEM((K1,), i32),
        pltpu.VMEM((K1,), i32),
        pltpu.VMEM((K1,), i32),
        pltpu.VMEM((K1,), i32),
        pltpu.VMEM((K1, C), f32),
        pltpu.VMEM((K1, C), f32),
        [pltpu.SemaphoreType.DMA] * 2,
        [pltpu.SemaphoreType.DMA] * 2,
    ],
)
def _sc_nsum(x_hbm, src_hbm, dst_hbm, zrow_hbm,
             nsum_out, nsum_sh, sidx0, sidx1, didx0, didx1, rows0, rows1,
             semA, semW):
  sidx = [sidx0, sidx1]
  didx = [didx0, didx1]
  rows = [rows0, rows1]
  c = lax.axis_index("c")
  s = lax.axis_index("s")
  pltpu.sync_copy(zrow_hbm, nsum_sh.at[pl.ds(s * STRIPE, STRIPE), :])
  plsc.subcore_barrier()

  fast = c == FAST_CORE
  base = s * PAIR_RT + jnp.where(fast, 0, RT1_F)
  iters = jnp.where(fast, RT1_F // K1, RT1_S // K1)

  def load_and_gather(i, b):
    off = pl.multiple_of(base + i * K1, 8)
    pltpu.sync_copy(src_hbm.at[pl.ds(off, K1)], sidx[b])
    pltpu.sync_copy(dst_hbm.at[pl.ds(off, K1)], didx[b])
    pltpu.async_copy(x_hbm.at[sidx[b]], rows[b], semA[b])

  load_and_gather(0, 0)
  load_and_gather(1, 1)

  def step(t, carry):
    for b in range(2):
      i = t * 2 + b
      pltpu.make_async_copy(x_hbm.at[sidx[b]], rows[b], semA[b]).wait()
      pltpu.async_copy(rows[b], nsum_sh.at[didx[b]], semW[b], add=True)
      pltpu.make_async_copy(rows[b], nsum_sh.at[didx[b]], semW[b]).wait()

      @pl.when(i + 2 < iters)
      def _():
        load_and_gather(i + 2, b)
    return carry

  lax.fori_loop(0, iters // 2, step, 0)
  plsc.subcore_barrier()
  sl = pl.ds(s * STRIPE, STRIPE)
  pltpu.sync_copy(nsum_sh.at[sl, :], nsum_out.at[c, sl, :])


@functools.partial(
    pl.kernel,
    out_type=jax.ShapeDtypeStruct((NC, N_PAD, C), f32),
    mesh=_mesh,
    scratch_types=[
        pltpu.VMEM_SHARED((N_PAD, C), f32),
        pltpu.VMEM((KC,), i32),
        pltpu.VMEM((KC,), i32),
        pltpu.VMEM((KC, C), f32),
        [pltpu.SemaphoreType.DMA] * 2,
        [pltpu.SemaphoreType.DMA] * 2,
    ],
)
def _sc_cnt(idx_hbm, ones_hbm, zrow_hbm, acc_out, acc_sh,
            didx0, didx1, ones_v, semI, semW):
  didx = [didx0, didx1]
  c = lax.axis_index("c")
  s = lax.axis_index("s")
  wid = s * NC + c
  pltpu.sync_copy(zrow_hbm, acc_sh.at[pl.ds(s * STRIPE, STRIPE), :])
  pltpu.sync_copy(ones_hbm, ones_v)
  plsc.subcore_barrier()

  base = wid * RT

  def load_idx(i, b):
    off = pl.multiple_of(base + i * KC, 8)
    pltpu.async_copy(idx_hbm.at[pl.ds(off, KC)], didx[b], semI[b])

  load_idx(0, 0)
  load_idx(1, 1)

  def step(t, carry):
    for b in range(2):
      i = t * 2 + b
      off = pl.multiple_of(base + i * KC, 8)
      pltpu.make_async_copy(
          idx_hbm.at[pl.ds(off, KC)], didx[b], semI[b]).wait()
      pltpu.async_copy(ones_v, acc_sh.at[didx[b]], semW[b], add=True)
      pltpu.make_async_copy(ones_v, acc_sh.at[didx[b]], semW[b]).wait()

      @pl.when(i + 2 < ITERSC)
      def _():
        load_idx(i + 2, b)
    return carry

  lax.fori_loop(0, ITERSC // 2, step, 0)
  plsc.subcore_barrier()
  sl = pl.ds(s * STRIPE, STRIPE)
  pltpu.sync_copy(acc_sh.at[sl, :], acc_out.at[c, sl, :])


@functools.partial(
    pl.kernel,
    out_type=jax.ShapeDtypeStruct((R_PAD, C), f32),
    mesh=_mesh,
    scratch_types=[
        pltpu.VMEM((RT3_F,), i32),
        pltpu.VMEM((RT3_F,), i32),
        pltpu.VMEM((K3, C), f32),
        pltpu.VMEM((K3, C), f32),
        [pltpu.SemaphoreType.DMA] * 2,
        [pltpu.SemaphoreType.DMA] * 2,
        [pltpu.SemaphoreType.DMA] * 2,
    ],
)
def _sc_gather(a_hbm, b_hbm, src_hbm, dst_hbm, p_out,
               sidx, didx, pbuf0, pbuf1, semA, semB, semS):
  pbuf = [pbuf0, pbuf1]
  c = lax.axis_index("c")
  s = lax.axis_index("s")
  fast = c == FAST_CORE
  base = pl.multiple_of(s * PAIR_RT + jnp.where(fast, 0, RT3_F), 8)
  iters = jnp.where(fast, RT3_F // K3, RT3_S // K3)
  pltpu.sync_copy(src_hbm.at[pl.ds(base, RT3_S)], sidx.at[pl.ds(0, RT3_S)])
  pltpu.sync_copy(dst_hbm.at[pl.ds(base, RT3_S)], didx.at[pl.ds(0, RT3_S)])

  @pl.when(fast)
  def _():
    tail = pl.multiple_of(base + RT3_S, 8)
    pltpu.sync_copy(src_hbm.at[pl.ds(tail, RT3_F - RT3_S)],
                    sidx.at[pl.ds(RT3_S, RT3_F - RT3_S)])
    pltpu.sync_copy(dst_hbm.at[pl.ds(tail, RT3_F - RT3_S)],
                    didx.at[pl.ds(RT3_S, RT3_F - RT3_S)])

  def gather_a(i, b):
    pltpu.async_copy(a_hbm.at[sidx.at[pl.ds(i * K3, K3)]], pbuf[b], semA[b])

  gather_a(0, 0)
  gather_a(1, 1)

  def step(t, carry):
    for b in range(2):
      i = t * 2 + b
      isl = pl.ds(i * K3, K3)
      pltpu.make_async_copy(
          a_hbm.at[sidx.at[isl]], pbuf[b], semA[b]).wait()
      pltpu.async_copy(b_hbm.at[didx.at[isl]], pbuf[b], semB[b], add=True)
      pltpu.make_async_copy(
          b_hbm.at[didx.at[isl]], pbuf[b], semB[b]).wait()
      off = pl.multiple_of(base + i * K3, 8)
      pltpu.async_copy(pbuf[b], p_out.at[pl.ds(off, K3), :], semS[b])

      @pl.when(i + 2 < iters)
      def _():
        pltpu.make_async_copy(
            pbuf[b], p_out.at[pl.ds(off, K3), :], semS[b]).wait()
        gather_a(i + 2, b)
    return carry

  lax.fori_loop(0, iters // 2, step, 0)
  for b in range(2):
    i = iters - 2 + b
    off = pl.multiple_of(base + i * K3, 8)
    pltpu.make_async_copy(pbuf[b], p_out.at[pl.ds(off, K3), :], semS[b]).wait()


def _dot(a, b):
  return lax.dot_general(a, b, (((1,), (0,)), ((), ())),
                         preferred_element_type=f32)


def _fold_body(nsum_p_ref, cs_ref, cd_ref, x_ref, g1_ref, bb1_ref, W1_ref,
               b1_ref, a_ref, b_ref):
  nsum = nsum_p_ref[0] + nsum_p_ref[1]
  cs = cs_ref[0] + cs_ref[1]
  cd = cd_ref[0] + cd_ref[1]
  x = x_ref[...]
  Rf = f32(R)
  sum_g = jnp.sum(cs * x, axis=0, keepdims=True)
  sumsq_g = jnp.sum(cs * (x * x), axis=0, keepdims=True)
  sum_n = jnp.sum(cd * nsum, axis=0, keepdims=True)
  sumsq_n = jnp.sum(cd * (nsum * nsum), axis=0, keepdims=True)
  m_g = sum_g / Rf
  v_g = sumsq_g / Rf - m_g * m_g
  m_n = sum_n / Rf
  v_n = sumsq_n / Rf - m_n * m_n
  g1 = g1_ref[...]
  bb1 = bb1_ref[...]
  s_g = g1[:, :C] * lax.rsqrt(v_g + EPS)
  s_n = g1[:, C:] * lax.rsqrt(v_n + EPS)
  c_g = bb1[:, :C] - m_g * s_g
  c_n = bb1[:, C:] - m_n * s_n
  W1t = W1_ref[:C, :]
  W1b = W1_ref[C:, :]
  b1p = b1_ref[...] + _dot(c_g, W1t) + _dot(c_n, W1b)
  a_ref[...] = _dot(x * s_g, W1t) + b1p
  b_ref[...] = _dot(nsum * s_n, W1b)


_tc_fold = pl.pallas_call(
    _fold_body,
    out_shape=(jax.ShapeDtypeStruct((N_PAD, C), f32),
               jax.ShapeDtypeStruct((N_PAD, C), f32)),
)


def _stats_body(p_ref, o_ref, h_ref):
  i = pl.program_id(0)

  @pl.when(i == 0)
  def _():
    o_ref[...] = jnp.zeros_like(o_ref)

  h = jnp.maximum(p_ref[...], 0.0)
  o_ref[0:1, :] += jnp.sum(h, axis=0, keepdims=True)
  o_ref[1:2, :] += jnp.sum(h * h, axis=0, keepdims=True)
  h_ref[...] = h.astype(jnp.bfloat16)


_tc_stats = pl.pallas_call(
    _stats_body,
    grid=(NBLK,),
    in_specs=[pl.BlockSpec((BR, C), lambda i: (i, 0))],
    out_specs=(pl.BlockSpec((2, C), lambda i: (0, 0)),
               pl.BlockSpec((BR, C), lambda i: (i, 0))),
    out_shape=(jax.ShapeDtypeStruct((2, C), f32),
               jax.ShapeDtypeStruct((R, C), jnp.bfloat16)),
)


def _final_body(h_ref, st_ref, g2_ref, bb2_ref, W2_ref, b2_ref, o_ref):
  Rf = f32(R)
  m2 = st_ref[0:1, :] / Rf
  v2 = st_ref[1:2, :] / Rf - m2 * m2
  s2 = g2_ref[...] * lax.rsqrt(v2 + EPS)
  c2 = bb2_ref[...] - m2 * s2
  h = h_ref[...].astype(f32)
  hn = h * s2 + c2
  o_ref[...] = jnp.maximum(_dot(hn, W2_ref[...]) + b2_ref[...], 0.0)


_tc_final = pl.pallas_call(
    _final_body,
    grid=(NBLK,),
    in_specs=[
        pl.BlockSpec((BR, C), lambda i: (i, 0)),
        pl.BlockSpec((2, C), lambda i: (0, 0)),
        pl.BlockSpec((1, C), lambda i: (0, 0)),
        pl.BlockSpec((1, C), lambda i: (0, 0)),
        pl.BlockSpec((C, C), lambda i: (0, 0)),
        pl.BlockSpec((1, C), lambda i: (0, 0)),
    ],
    out_specs=pl.BlockSpec((BR, C), lambda i: (i, 0)),
    out_shape=jax.ShapeDtypeStruct((R, C), f32),
)


@jax.jit
def kernel(x, edge_index, bn1_g, bn1_b, W1, b1, bn2_g, bn2_b, W2, b2):
  src = edge_index[0].astype(i32)
  dst = edge_index[1].astype(i32)
  self_idx = jnp.arange(N, dtype=i32)
  padv = jnp.full((R_PAD - R,), N, i32)
  src_all = jnp.concatenate([src, self_idx, padv])
  dst_all = jnp.concatenate([dst, self_idx, padv])
  x_pad = jnp.pad(x, ((0, N_PAD - N), (0, 0)))

  zrow = jnp.zeros((STRIPE, C), f32)
  ones_h = jnp.ones((KC, C), f32)

  nsum_parts = _sc_nsum(x_pad, src_all, dst_all, zrow)
  cs_parts = _sc_cnt(src_all, ones_h, zrow)
  cd_parts = _sc_cnt(dst_all, ones_h, zrow)
  cs = cs_parts[:, :, 0:1]
  cd = cd_parts[:, :, 0:1]

  A_pad, B_pad = _tc_fold(
      nsum_parts, cs, cd, x_pad,
      bn1_g.reshape(1, 2 * C), bn1_b.reshape(1, 2 * C),
      W1, b1.reshape(1, C))

  P = _sc_gather(A_pad, B_pad, src_all, dst_all)

  stats, H = _tc_stats(P)

  out = _tc_final(H, stats, bn2_g.reshape(1, C), bn2_b.reshape(1, C),
                  W2, b2.reshape(1, C))
  return out


# speedup vs baseline: 1.0611x; 1.0423x over previous
import functools

import jax
import jax.numpy as jnp
from jax import lax
from jax.experimental import pallas as pl
from jax.experimental.pallas import tpu as pltpu
from jax.experimental.pallas import tpu_sc as plsc

N = 10000
E = 320000
C = 128
R = E + N
NC, NS, L = 2, 16, 16
NW = NC * NS
N_PAD = 10240
STRIPE = N_PAD // NS
RT = 10400
R_PAD = RT * NW
K1 = 104
ITERS1 = RT // K1
K3 = 400
ITERS3 = RT // K3
FAST_CORE = 0
PAIR_RT = 2 * RT
RT1_F, RT1_S = 13520, 7280
RT3_F, RT3_S = 13600, 7200
KC = 104
ITERSC = RT // KC
CEPS = 1.0 / 1024.0
BR = 2000
NBLK = R // BR
EPS = 1e-5

_mesh = plsc.VectorSubcoreMesh(
    core_axis_name="c", subcore_axis_name="s", num_cores=NC, num_subcores=NS)

f32 = jnp.float32
i32 = jnp.int32


@functools.partial(
    pl.kernel,
    out_type=jax.ShapeDtypeStruct((NC, N_PAD, C), f32),
    mesh=_mesh,
    scratch_types=[
        pltpu.VMEM_SHARED((N_PAD, C), f32),
        pltpu.VMEM((K1,), i32),
        pltpu.VMEM((K1,), i32),
        pltpu.VMEM((K1,), i32),
        pltpu.VMEM((K1,), i32),
        pltpu.VMEM((K1, C), f32),
        pltpu.VMEM((K1, C), f32),
        [pltpu.SemaphoreType.DMA] * 2,
        [pltpu.SemaphoreType.DMA] * 2,
    ],
)
def _sc_nsum(x_hbm, src_hbm, dst_hbm, zrow_hbm,
             nsum_out, nsum_sh, sidx0, sidx1, didx0, didx1, rows0, rows1,
             semA, semW):
  sidx = [sidx0, sidx1]
  didx = [didx0, didx1]
  rows = [rows0, rows1]
  c = lax.axis_index("c")
  s = lax.axis_index("s")
  pltpu.sync_copy(zrow_hbm, nsum_sh.at[pl.ds(s * STRIPE, STRIPE), :])
  plsc.subcore_barrier()

  fast = c == FAST_CORE
  base = s * PAIR_RT + jnp.where(fast, 0, RT1_F)
  iters = jnp.where(fast, RT1_F // K1, RT1_S // K1)

  def load_and_gather(i, b):
    off = pl.multiple_of(base + i * K1, 8)
    pltpu.sync_copy(src_hbm.at[pl.ds(off, K1)], sidx[b])
    pltpu.sync_copy(dst_hbm.at[pl.ds(off, K1)], didx[b])
    pltpu.async_copy(x_hbm.at[sidx[b]], rows[b], semA[b])

  load_and_gather(0, 0)
  load_and_gather(1, 1)

  def step(t, carry):
    for b in range(2):
      i = t * 2 + b
      pltpu.make_async_copy(x_hbm.at[sidx[b]], rows[b], semA[b]).wait()
      pltpu.async_copy(rows[b], nsum_sh.at[didx[b]], semW[b], add=True)
      pltpu.make_async_copy(rows[b], nsum_sh.at[didx[b]], semW[b]).wait()

      @pl.when(i + 2 < iters)
      def _():
        load_and_gather(i + 2, b)
    return carry

  lax.fori_loop(0, iters // 2, step, 0)
  plsc.subcore_barrier()
  sl = pl.ds(s * STRIPE, STRIPE)
  pltpu.sync_copy(nsum_sh.at[sl, :], nsum_out.at[c, sl, :])


@functools.partial(
    pl.kernel,
    out_type=jax.ShapeDtypeStruct((NC, N_PAD, C), f32),
    mesh=_mesh,
    scratch_types=[
        pltpu.VMEM_SHARED((N_PAD, C), f32),
        pltpu.VMEM((KC,), i32),
        pltpu.VMEM((KC,), i32),
        pltpu.VMEM((KC,), i32),
        pltpu.VMEM((KC,), i32),
        pltpu.VMEM((KC, C), f32),
        pltpu.VMEM((KC, C), f32),
        [pltpu.SemaphoreType.DMA] * 2,
        [pltpu.SemaphoreType.DMA] * 2,
        [pltpu.SemaphoreType.DMA] * 2,
        [pltpu.SemaphoreType.DMA] * 2,
    ],
)
def _sc_cnt(src_hbm, dst_hbm, ones_hbm, eps_hbm, zrow_hbm, acc_out, acc_sh,
            sidx0, sidx1, didx0, didx1, ones_v, eps_v,
            semI, semJ, semW, semV):
  sidx = [sidx0, sidx1]
  didx = [didx0, didx1]
  c = lax.axis_index("c")
  s = lax.axis_index("s")
  wid = s * NC + c
  pltpu.sync_copy(zrow_hbm, acc_sh.at[pl.ds(s * STRIPE, STRIPE), :])
  pltpu.sync_copy(ones_hbm, ones_v)
  pltpu.sync_copy(eps_hbm, eps_v)
  plsc.subcore_barrier()

  base = wid * RT

  def load_idx(i, b):
    off = pl.multiple_of(base + i * KC, 8)
    pltpu.async_copy(src_hbm.at[pl.ds(off, KC)], sidx[b], semI[b])
    pltpu.async_copy(dst_hbm.at[pl.ds(off, KC)], didx[b], semJ[b])

  load_idx(0, 0)
  load_idx(1, 1)

  def step(t, carry):
    for b in range(2):
      i = t * 2 + b
      off = pl.multiple_of(base + i * KC, 8)
      pltpu.make_async_copy(
          src_hbm.at[pl.ds(off, KC)], sidx[b], semI[b]).wait()
      pltpu.make_async_copy(
          dst_hbm.at[pl.ds(off, KC)], didx[b], semJ[b]).wait()
      pltpu.async_copy(ones_v, acc_sh.at[sidx[b]], semW[b], add=True)
      pltpu.async_copy(eps_v, acc_sh.at[didx[b]], semV[b], add=True)
      pltpu.make_async_copy(ones_v, acc_sh.at[sidx[b]], semW[b]).wait()
      pltpu.make_async_copy(eps_v, acc_sh.at[didx[b]], semV[b]).wait()

      @pl.when(i + 2 < ITERSC)
      def _():
        load_idx(i + 2, b)
    return carry

  lax.fori_loop(0, ITERSC // 2, step, 0)
  plsc.subcore_barrier()
  sl = pl.ds(s * STRIPE, STRIPE)
  pltpu.sync_copy(acc_sh.at[sl, :], acc_out.at[c, sl, :])


@functools.partial(
    pl.kernel,
    out_type=jax.ShapeDtypeStruct((R_PAD, C), f32),
    mesh=_mesh,
    scratch_types=[
        pltpu.VMEM((RT3_F,), i32),
        pltpu.VMEM((RT3_F,), i32),
        pltpu.VMEM((K3, C), f32),
        pltpu.VMEM((K3, C), f32),
        [pltpu.SemaphoreType.DMA] * 2,
        [pltpu.SemaphoreType.DMA] * 2,
        [pltpu.SemaphoreType.DMA] * 2,
    ],
)
def _sc_gather(a_hbm, b_hbm, src_hbm, dst_hbm, p_out,
               sidx, didx, pbuf0, pbuf1, semA, semB, semS):
  pbuf = [pbuf0, pbuf1]
  c = lax.axis_index("c")
  s = lax.axis_index("s")
  fast = c == FAST_CORE
  base = pl.multiple_of(s * PAIR_RT + jnp.where(fast, 0, RT3_F), 8)
  iters = jnp.where(fast, RT3_F // K3, RT3_S // K3)
  pltpu.sync_copy(src_hbm.at[pl.ds(base, RT3_S)], sidx.at[pl.ds(0, RT3_S)])
  pltpu.sync_copy(dst_hbm.at[pl.ds(base, RT3_S)], didx.at[pl.ds(0, RT3_S)])

  @pl.when(fast)
  def _():
    tail = pl.multiple_of(base + RT3_S, 8)
    pltpu.sync_copy(src_hbm.at[pl.ds(tail, RT3_F - RT3_S)],
                    sidx.at[pl.ds(RT3_S, RT3_F - RT3_S)])
    pltpu.sync_copy(dst_hbm.at[pl.ds(tail, RT3_F - RT3_S)],
                    didx.at[pl.ds(RT3_S, RT3_F - RT3_S)])

  def gather_a(i, b):
    pltpu.async_copy(a_hbm.at[sidx.at[pl.ds(i * K3, K3)]], pbuf[b], semA[b])

  gather_a(0, 0)
  gather_a(1, 1)

  def step(t, carry):
    for b in range(2):
      i = t * 2 + b
      isl = pl.ds(i * K3, K3)
      pltpu.make_async_copy(
          a_hbm.at[sidx.at[isl]], pbuf[b], semA[b]).wait()
      pltpu.async_copy(b_hbm.at[didx.at[isl]], pbuf[b], semB[b], add=True)
      pltpu.make_async_copy(
          b_hbm.at[didx.at[isl]], pbuf[b], semB[b]).wait()
      off = pl.multiple_of(base + i * K3, 8)
      pltpu.async_copy(pbuf[b], p_out.at[pl.ds(off, K3), :], semS[b])

      @pl.when(i + 2 < iters)
      def _():
        pltpu.make_async_copy(
            pbuf[b], p_out.at[pl.ds(off, K3), :], semS[b]).wait()
        gather_a(i + 2, b)
    return carry

  lax.fori_loop(0, iters // 2, step, 0)
  for b in range(2):
    i = iters - 2 + b
    off = pl.multiple_of(base + i * K3, 8)
    pltpu.make_async_copy(pbuf[b], p_out.at[pl.ds(off, K3), :], semS[b]).wait()


def _dot(a, b):
  return lax.dot_general(a, b, (((1,), (0,)), ((), ())),
                         preferred_element_type=f32)


def _fold_body(nsum_p_ref, cnt_ref, x_ref, g1_ref, bb1_ref, W1_ref,
               b1_ref, a_ref, b_ref):
  nsum = nsum_p_ref[0] + nsum_p_ref[1]
  acc = cnt_ref[0] + cnt_ref[1]
  cs = jnp.floor(acc)
  cd = jnp.floor((acc - cs) / CEPS + 0.5)
  x = x_ref[...]
  Rf = f32(R)
  sum_g = jnp.sum(cs * x, axis=0, keepdims=True)
  sumsq_g = jnp.sum(cs * (x * x), axis=0, keepdims=True)
  sum_n = jnp.sum(cd * nsum, axis=0, keepdims=True)
  sumsq_n = jnp.sum(cd * (nsum * nsum), axis=0, keepdims=True)
  m_g = sum_g / Rf
  v_g = sumsq_g / Rf - m_g * m_g
  m_n = sum_n / Rf
  v_n = sumsq_n / Rf - m_n * m_n
  g1 = g1_ref[...]
  bb1 = bb1_ref[...]
  s_g = g1[:, :C] * lax.rsqrt(v_g + EPS)
  s_n = g1[:, C:] * lax.rsqrt(v_n + EPS)
  c_g = bb1[:, :C] - m_g * s_g
  c_n = bb1[:, C:] - m_n * s_n
  W1t = W1_ref[:C, :]
  W1b = W1_ref[C:, :]
  b1p = b1_ref[...] + _dot(c_g, W1t) + _dot(c_n, W1b)
  a_ref[...] = _dot(x * s_g, W1t) + b1p
  b_ref[...] = _dot(nsum * s_n, W1b)


_tc_fold = pl.pallas_call(
    _fold_body,
    out_shape=(jax.ShapeDtypeStruct((N_PAD, C), f32),
               jax.ShapeDtypeStruct((N_PAD, C), f32)),
)


def _stats_body(p_ref, o_ref, h_ref):
  i = pl.program_id(0)

  @pl.when(i == 0)
  def _():
    o_ref[...] = jnp.zeros_like(o_ref)

  h = jnp.maximum(p_ref[...], 0.0)
  o_ref[0:1, :] += jnp.sum(h, axis=0, keepdims=True)
  o_ref[1:2, :] += jnp.sum(h * h, axis=0, keepdims=True)
  h_ref[...] = h.astype(jnp.bfloat16)


_tc_stats = pl.pallas_call(
    _stats_body,
    grid=(NBLK,),
    in_specs=[pl.BlockSpec((BR, C), lambda i: (i, 0))],
    out_specs=(pl.BlockSpec((2, C), lambda i: (0, 0)),
               pl.BlockSpec((BR, C), lambda i: (i, 0))),
    out_shape=(jax.ShapeDtypeStruct((2, C), f32),
               jax.ShapeDtypeStruct((R, C), jnp.bfloat16)),
)


def _final_body(h_ref, st_ref, g2_ref, bb2_ref, W2_ref, b2_ref, o_ref):
  Rf = f32(R)
  m2 = st_ref[0:1, :] / Rf
  v2 = st_ref[1:2, :] / Rf - m2 * m2
  s2 = g2_ref[...] * lax.rsqrt(v2 + EPS)
  c2 = bb2_ref[...] - m2 * s2
  h = h_ref[...].astype(f32)
  hn = h * s2 + c2
  o_ref[...] = jnp.maximum(_dot(hn, W2_ref[...]) + b2_ref[...], 0.0)


_tc_final = pl.pallas_call(
    _final_body,
    grid=(NBLK,),
    in_specs=[
        pl.BlockSpec((BR, C), lambda i: (i, 0)),
        pl.BlockSpec((2, C), lambda i: (0, 0)),
        pl.BlockSpec((1, C), lambda i: (0, 0)),
        pl.BlockSpec((1, C), lambda i: (0, 0)),
        pl.BlockSpec((C, C), lambda i: (0, 0)),
        pl.BlockSpec((1, C), lambda i: (0, 0)),
    ],
    out_specs=pl.BlockSpec((BR, C), lambda i: (i, 0)),
    out_shape=jax.ShapeDtypeStruct((R, C), f32),
)


@jax.jit
def kernel(x, edge_index, bn1_g, bn1_b, W1, b1, bn2_g, bn2_b, W2, b2):
  src = edge_index[0].astype(i32)
  dst = edge_index[1].astype(i32)
  self_idx = jnp.arange(N, dtype=i32)
  padv = jnp.full((R_PAD - R,), N, i32)
  src_all = jnp.concatenate([src, self_idx, padv])
  dst_all = jnp.concatenate([dst, self_idx, padv])
  x_pad = jnp.pad(x, ((0, N_PAD - N), (0, 0)))

  zrow = jnp.zeros((STRIPE, C), f32)
  ones_h = jnp.ones((KC, C), f32)

  eps_h = jnp.full((KC, C), CEPS, f32)
  nsum_parts = _sc_nsum(x_pad, src_all, dst_all, zrow)
  cnt_parts = _sc_cnt(src_all, dst_all, ones_h, eps_h, zrow)
  cnt = cnt_parts[:, :, 0:1]

  A_pad, B_pad = _tc_fold(
      nsum_parts, cnt, x_pad,
      bn1_g.reshape(1, 2 * C), bn1_b.reshape(1, 2 * C),
      W1, b1.reshape(1, C))

  P = _sc_gather(A_pad, B_pad, src_all, dst_all)

  stats, H = _tc_stats(P)

  out = _tc_final(H, stats, bn2_g.reshape(1, C), bn2_b.reshape(1, C),
                  W2, b2.reshape(1, C))
  return out
